# Initial kernel scaffold; baseline (speedup 1.0000x reference)
import jax
import jax.numpy as jnp
from jax.experimental import pallas as pl


def kernel(z, codebook):
    raise NotImplementedError("write your pallas kernel here")


# speedup vs baseline: 1.0469x; 1.0469x over previous
import functools

import jax
import jax.numpy as jnp
from jax import lax
from jax.experimental import pallas as pl
from jax.experimental.pallas import tpu as pltpu
from jax.experimental.pallas import tpu_sc as plsc

_B, _T, _D, _K = 8, 1024, 256, 8192
_BETA = 0.25
_N = _B * _T
_NT = 256
_CHUNK = 2048


def _vq_body(z_ref, cb_ref, zn_ref, cnorm_ref, idx_ref, loss_ref, acc_ref):
    n = pl.program_id(0)
    z = z_ref[...]
    cb = cb_ref[...]
    zn = zn_ref[...]
    cnorm = cnorm_ref[...]
    dots = lax.dot_general(
        z.astype(jnp.bfloat16), cb.astype(jnp.bfloat16),
        (((1,), (1,)), ((), ())), preferred_element_type=jnp.float32,
    )
    d = (zn - 2.0 * dots) + cnorm

    acc_v = None
    for c in range(_K // _CHUNK):
        dc = d[:, c * _CHUNK:(c + 1) * _CHUNK]
        v = jnp.min(dc, axis=1)
        i = jnp.argmin(dc, axis=1).astype(jnp.int32) + c * _CHUNK
        if acc_v is None:
            acc_v, acc_i = v, i
            exact_min = v
        else:
            take = v < acc_v
            acc_v = jnp.where(take, v, acc_v)
            acc_i = jnp.where(take, i, acc_i)
            exact_min = jnp.minimum(exact_min, v)
        acc_v = acc_v.astype(jnp.bfloat16).astype(jnp.float32)

    idx_ref[...] = acc_i

    @pl.when(n == 0)
    def _():
        acc_ref[0, 0] = 0.0

    acc_ref[0, 0] += jnp.sum(exact_min)

    @pl.when(n == pl.num_programs(0) - 1)
    def _():
        loss_ref[0, 0] = acc_ref[0, 0] * ((1.0 + _BETA) / (_N * _D))


def _vq_argmin(flat, codebook, zn, cnorm, interpret=False):
    grid = _N // _NT
    return pl.pallas_call(
        _vq_body,
        grid=(grid,),
        in_specs=[
            pl.BlockSpec((_NT, _D), lambda n: (n, 0)),
            pl.BlockSpec((_K, _D), lambda n: (0, 0)),
            pl.BlockSpec((_NT, 1), lambda n: (n, 0)),
            pl.BlockSpec((1, _K), lambda n: (0, 0)),
        ],
        out_specs=[
            pl.BlockSpec((_NT,), lambda n: (n,)),
            pl.BlockSpec(memory_space=pltpu.SMEM),
        ],
        out_shape=[
            jax.ShapeDtypeStruct((_N,), jnp.int32),
            jax.ShapeDtypeStruct((1, 1), jnp.float32),
        ],
        scratch_shapes=[pltpu.SMEM((1, 1), jnp.float32)],
        interpret=interpret,
    )(flat, codebook, zn, cnorm)


@functools.cache
def _make_sc_gather():
    info = plsc.get_sparse_core_info()
    nw = info.num_cores * info.num_subcores
    b_per_w = _N // nw
    mesh = plsc.VectorSubcoreMesh(core_axis_name="c", subcore_axis_name="s")

    @functools.partial(
        pl.kernel,
        mesh=mesh,
        out_type=jax.ShapeDtypeStruct((_N, _D), jnp.float32),
        scratch_types=[
            pltpu.VMEM((b_per_w,), jnp.int32),
            pltpu.VMEM((b_per_w, _D), jnp.float32),
            pltpu.SemaphoreType.DMA,
        ],
    )
    def gather_k(table_hbm, idx_hbm, out_hbm, idx_v, rows_v, sem):
        wid = lax.axis_index("s") * info.num_cores + lax.axis_index("c")
        base = wid * b_per_w
        pltpu.sync_copy(idx_hbm.at[pl.ds(base, b_per_w)], idx_v)
        pltpu.async_copy(table_hbm.at[idx_v], rows_v, sem).wait()
        pltpu.sync_copy(rows_v, out_hbm.at[pl.ds(base, b_per_w)])

    return gather_k


def kernel(z, codebook):
    flat = z.reshape(-1, _D)
    zn = jnp.sum(z * z, axis=2).reshape(_N, 1)
    cnorm = jnp.sum(codebook * codebook, axis=1)[None, :]
    idx, loss = _vq_argmin(flat, codebook, zn, cnorm)
    quantized = _make_sc_gather()(codebook, idx)
    q_st = quantized.reshape(z.shape)
    return q_st, idx.reshape(_B, _T), loss[0, 0]


# speedup vs baseline: 1.1030x; 1.0536x over previous
import functools

import jax
import jax.numpy as jnp
from jax import lax
from jax.experimental import pallas as pl
from jax.experimental.pallas import tpu as pltpu
from jax.experimental.pallas import tpu_sc as plsc

_B, _T, _D, _K = 8, 1024, 256, 8192
_BETA = 0.25
_N = _B * _T
_NT = 256
_CHUNK = 2048


def _vq_body(z_ref, cb_ref, zn_ref, cnorm_ref, idx_ref, loss_ref, acc_ref):
    n = pl.program_id(0)
    z = z_ref[...]
    cb = cb_ref[...]
    zn = zn_ref[...]
    cnorm = cnorm_ref[...]
    dots2 = lax.dot_general(
        (-2.0 * z).astype(jnp.bfloat16), cb.astype(jnp.bfloat16),
        (((1,), (1,)), ((), ())), preferred_element_type=jnp.float32,
    )
    d = (zn + dots2) + cnorm

    acc_v = None
    for c in range(_K // _CHUNK):
        dc = d[:, c * _CHUNK:(c + 1) * _CHUNK]
        v = jnp.min(dc, axis=1)
        i = jnp.argmin(dc, axis=1).astype(jnp.int32) + c * _CHUNK
        if acc_v is None:
            acc_v, acc_i = v, i
            exact_min = v
        else:
            take = v < acc_v
            acc_v = jnp.where(take, v, acc_v)
            acc_i = jnp.where(take, i, acc_i)
            exact_min = jnp.minimum(exact_min, v)
        acc_v = acc_v.astype(jnp.bfloat16).astype(jnp.float32)

    idx_ref[...] = acc_i

    @pl.when(n == 0)
    def _():
        acc_ref[0, 0] = 0.0

    acc_ref[0, 0] += jnp.sum(exact_min)

    @pl.when(n == pl.num_programs(0) - 1)
    def _():
        loss_ref[0, 0] = acc_ref[0, 0] * ((1.0 + _BETA) / (_N * _D))


def _vq_argmin(flat, codebook, zn, cnorm, interpret=False):
    grid = _N // _NT
    return pl.pallas_call(
        _vq_body,
        grid=(grid,),
        in_specs=[
            pl.BlockSpec((_NT, _D), lambda n: (n, 0)),
            pl.BlockSpec((_K, _D), lambda n: (0, 0)),
            pl.BlockSpec((_NT, 1), lambda n: (n, 0)),
            pl.BlockSpec((1, _K), lambda n: (0, 0)),
        ],
        out_specs=[
            pl.BlockSpec((_NT,), lambda n: (n,)),
            pl.BlockSpec(memory_space=pltpu.SMEM),
        ],
        out_shape=[
            jax.ShapeDtypeStruct((_N,), jnp.int32),
            jax.ShapeDtypeStruct((1, 1), jnp.float32),
        ],
        scratch_shapes=[pltpu.SMEM((1, 1), jnp.float32)],
        interpret=interpret,
    )(flat, codebook, zn, cnorm)


@functools.cache
def _make_sc_gather():
    info = plsc.get_sparse_core_info()
    nw = info.num_cores * info.num_subcores
    b_per_w = _N // nw
    mesh = plsc.VectorSubcoreMesh(core_axis_name="c", subcore_axis_name="s")

    @functools.partial(
        pl.kernel,
        mesh=mesh,
        out_type=jax.ShapeDtypeStruct((_N, _D), jnp.float32),
        scratch_types=[
            pltpu.VMEM((b_per_w,), jnp.int32),
            pltpu.VMEM((b_per_w, _D), jnp.float32),
            pltpu.SemaphoreType.DMA,
        ],
    )
    def gather_k(table_hbm, idx_hbm, out_hbm, idx_v, rows_v, sem):
        wid = lax.axis_index("s") * info.num_cores + lax.axis_index("c")
        base = wid * b_per_w
        pltpu.sync_copy(idx_hbm.at[pl.ds(base, b_per_w)], idx_v)
        pltpu.async_copy(table_hbm.at[idx_v], rows_v, sem).wait()
        pltpu.sync_copy(rows_v, out_hbm.at[pl.ds(base, b_per_w)])

    return gather_k


def kernel(z, codebook):
    flat = z.reshape(-1, _D)
    zn = jnp.sum(z * z, axis=2).reshape(_N, 1)
    cnorm = jnp.sum(codebook * codebook, axis=1)[None, :]
    idx, loss = _vq_argmin(flat, codebook, zn, cnorm)
    quantized = _make_sc_gather()(codebook, idx)
    q_st = quantized.reshape(z.shape)
    return q_st, idx.reshape(_B, _T), loss[0, 0]
